# baseline (speedup 1.0000x reference)
import functools

import jax
import jax.numpy as jnp
from jax import lax
from jax.experimental import pallas as pl
from jax.experimental.pallas import tpu as pltpu
from jax.experimental.pallas import tpu_sc as plsc

N = 10000
E = 320000
H = 128
G = 64
NC = 2
NS = 16
NW = NC * NS
CPW = H // NW
EPW = E // NW
CE = 4000
NCH = E // CE
L = 16

_MESH = dict(core_axis_name="c", subcore_axis_name="s", num_cores=NC,
             num_subcores=NS)


def _wid():
    return lax.axis_index("s") * NC + lax.axis_index("c")


def _deg_body(dst_hbm, degp_hbm, cnt, dbuf):
    w = _wid()
    zeros = jnp.zeros((L,), jnp.float32)
    ones = jnp.ones((L,), jnp.float32)

    @plsc.parallel_loop(0, N // L, unroll=8)
    def z(i):
        cnt[pl.ds(i * L, L)] = zeros

    pltpu.sync_copy(dst_hbm.at[pl.ds(w * EPW, EPW)], dbuf)

    @plsc.parallel_loop(0, EPW // L, unroll=8)
    def b(i):
        d16 = dbuf[pl.ds(i * L, L)]
        plsc.addupdate_scatter(cnt, [d16], ones)

    pltpu.sync_copy(cnt, degp_hbm.at[pl.ds(w * N, N)])


_deg_call = pl.kernel(
    _deg_body,
    out_type=jax.ShapeDtypeStruct((NW * N,), jnp.float32),
    mesh=plsc.VectorSubcoreMesh(**_MESH),
    compiler_params=pltpu.CompilerParams(needs_layout_passes=False),
    scratch_types=[
        pltpu.VMEM((N,), jnp.float32),
        pltpu.VMEM((EPW,), jnp.int32),
    ],
)


def _pre_body(degp, xT, Wg, Wl, bl2, wai2, waj2, ba2,
              dinv_o, yT_o, xlT_o, ai_o, aj_o):
    dot00 = lambda a, b: lax.dot_general(
        a, b, (((0,), (0,)), ((), ())), preferred_element_type=jnp.float32)
    deg = jnp.sum(degp[...], axis=0, keepdims=True) + 1.0
    dinv = lax.rsqrt(deg)
    dinv_o[...] = dinv
    xt = xT[...]
    yT_o[...] = dot00(Wg[...], xt) * dinv
    xlt = dot00(Wl[...], xt) + bl2[...]
    xlT_o[...] = xlt
    ai_o[...] = dot00(wai2[...], xlt) + ba2[...]
    aj_o[...] = dot00(waj2[...], xlt)


_pre_call = pl.pallas_call(
    _pre_body,
    out_shape=[
        jax.ShapeDtypeStruct((1, N), jnp.float32),
        jax.ShapeDtypeStruct((H, N), jnp.float32),
        jax.ShapeDtypeStruct((H, N), jnp.float32),
        jax.ShapeDtypeStruct((1, N), jnp.float32),
        jax.ShapeDtypeStruct((1, N), jnp.float32),
    ],
)


_OFF_Y = 0
_OFF_XL = H * N
_OFF_AI = 2 * H * N
_OFF_AJ = 2 * H * N + N
_OFF_POS = 2 * H * N + 2 * N
_OFF_WP1 = 2 * H * N + 5 * N
_TAB_SZ = _OFF_WP1 + H * 4 * L


def _main_body(ep_hbm, tab_hbm,
               gT_o, axl_o, ah1_o, s0_o,
               tab4, tab2, acc, s0acc, eb0, eb1, wv, sem0, sem1):
    w = _wid()
    w4 = w * CPW
    zeros = jnp.zeros((L,), jnp.float32)
    kns = [jnp.full((L,), k * N, jnp.int32) for k in range(4)]
    m0 = jnp.broadcast_to(w == 0, (L,))

    def zero_acc():
        @plsc.parallel_loop(0, CPW * N // L, unroll=8)
        def z(i):
            acc[pl.ds(i * L, L)] = zeros

    ebufs = (eb0, eb1)

    def issue(c, b, sem):
        pltpu.async_copy(ep_hbm.at[pl.ds(c * CE, CE)], ebufs[b], sem)

    def issue_guarded(c, b, sem):
        @pl.when(c < NCH)
        def _():
            issue(c, b, sem)

    def wait(b, sem):
        pltpu.make_async_copy(
            ep_hbm.at[pl.ds(0, CE)], ebufs[b], sem).wait()

    def edge_loop(process16):
        def inner(b):
            @plsc.parallel_loop(0, CE // L, unroll=8)
            def it(i):
                e16 = ebufs[b][pl.ds(i * L, L)]
                s16 = lax.shift_right_logical(e16, 14)
                d16 = jnp.bitwise_and(e16, 16383)
                process16(s16, d16)

        issue(0, 0, sem0)
        issue(1, 1, sem1)

        def outer(j, c):
            c0 = j * 2
            wait(0, sem0)
            inner(0)
            issue_guarded(c0 + 2, 0, sem0)
            wait(1, sem1)
            inner(1)
            issue_guarded(c0 + 3, 1, sem1)
            return c
        lax.fori_loop(0, NCH // 2, outer, 0)

    def att_of(s16, d16):
        av_i = plsc.load_gather(tab2, [d16])
        av_j = plsc.load_gather(tab2, [s16 + N])
        return 1.0 / (1.0 + jnp.exp(-(av_i + av_j)))

    pltpu.sync_copy(tab_hbm.at[pl.ds(_OFF_Y + w4 * N, CPW * N)], tab4)
    zero_acc()

    def p_a(s16, d16):
        for k in range(4):
            v = plsc.load_gather(tab4, [s16 + kns[k]])
            plsc.addupdate_scatter(acc, [d16 + kns[k]], v)
    edge_loop(p_a)
    pltpu.sync_copy(acc, gT_o.at[pl.ds(w4 * N, CPW * N)])

    pltpu.sync_copy(tab_hbm.at[pl.ds(_OFF_XL + w4 * N, CPW * N)], tab4)
    pltpu.sync_copy(tab_hbm.at[pl.ds(_OFF_AI, 2 * N)], tab2)
    zero_acc()

    def p_b1(s16, d16):
        att = att_of(s16, d16)
        for k in range(4):
            v = plsc.load_gather(tab4, [s16 + kns[k]]) * att
            plsc.addupdate_scatter(acc, [d16 + kns[k]], v)
    edge_loop(p_b1)
    pltpu.sync_copy(acc, axl_o.at[pl.ds(w4 * N, CPW * N)])

    pltpu.sync_copy(tab_hbm.at[pl.ds(_OFF_POS, 3 * N)], tab4.at[pl.ds(0, 3 * N)])
    pltpu.sync_copy(tab_hbm.at[pl.ds(_OFF_WP1 + w * 4 * L * 4, 4 * L * 4)], wv)
    zero_acc()

    @plsc.parallel_loop(0, N // L, unroll=8)
    def zs0(i):
        s0acc[pl.ds(i * L, L)] = zeros

    wvv = [[wv[pl.ds((k * 4 + j) * L, L)] for j in range(4)] for k in range(4)]

    def p_b2(s16, d16):
        att = att_of(s16, d16)
        dx = plsc.load_gather(tab4, [d16]) - plsc.load_gather(tab4, [s16])
        dy = plsc.load_gather(tab4, [d16 + N]) - plsc.load_gather(
            tab4, [s16 + N])
        dz = plsc.load_gather(tab4, [d16 + 2 * N]) - plsc.load_gather(
            tab4, [s16 + 2 * N])
        for k in range(4):
            h = jnp.maximum(
                dx * wvv[k][0] + dy * wvv[k][1] + dz * wvv[k][2] + wvv[k][3],
                0.0) * att
            plsc.addupdate_scatter(acc, [d16 + kns[k]], h)
        plsc.addupdate_scatter(s0acc, [d16], att, mask=m0)
    edge_loop(p_b2)
    pltpu.sync_copy(acc, ah1_o.at[pl.ds(w4 * N, CPW * N)])

    @pl.when(w == 0)
    def _():
        pltpu.sync_copy(s0acc, s0_o)


_main_call = pl.kernel(
    _main_body,
    out_type=[
        jax.ShapeDtypeStruct((H * N,), jnp.float32),
        jax.ShapeDtypeStruct((H * N,), jnp.float32),
        jax.ShapeDtypeStruct((H * N,), jnp.float32),
        jax.ShapeDtypeStruct((N,), jnp.float32),
    ],
    mesh=plsc.VectorSubcoreMesh(**_MESH),
    compiler_params=pltpu.CompilerParams(needs_layout_passes=False),
    scratch_types=[
        pltpu.VMEM((CPW * N,), jnp.float32),
        pltpu.VMEM((2 * N,), jnp.float32),
        pltpu.VMEM((CPW * N,), jnp.float32),
        pltpu.VMEM((N,), jnp.float32),
        pltpu.VMEM((CE,), jnp.int32),
        pltpu.VMEM((CE,), jnp.int32),
        pltpu.VMEM((4 * L * 4,), jnp.float32),
        pltpu.SemaphoreType.DMA,
        pltpu.SemaphoreType.DMA,
    ],
)


def _post_body(gT, axl, ah1, s02, yT, dinv, batch2, Wp2, bp22, bg2,
               Wf, bf2, W1, b12, W2, b22, out_o):
    dot00 = lambda a, b: lax.dot_general(
        a, b, (((0,), (0,)), ((), ())), preferred_element_type=jnp.float32)
    dv = dinv[...]
    globalT = jnp.maximum(dv * (gT[...] + yT[...]) + bg2[...], 0.0)
    localT = jnp.maximum(
        axl[...] + dot00(Wp2[...], ah1[...]) + bp22[...] * s02[...], 0.0)
    wf = Wf[...]
    fusedT = jnp.maximum(
        dot00(wf[:H], globalT) + dot00(wf[H:], localT) + bf2[...], 0.0)
    oh = (batch2[...] == lax.broadcasted_iota(jnp.int32, (N, G), 1)
          ).astype(jnp.float32)
    sumsT = jnp.dot(fusedT, oh, preferred_element_type=jnp.float32)
    cnts = jnp.sum(oh, axis=0, keepdims=True)
    pooledT = sumsT / jnp.maximum(cnts, 1.0)
    hT = jnp.maximum(dot00(W1[...], pooledT) + b12[...], 0.0)
    out_o[...] = dot00(W2[...], hT) + b22[...]


_post_call = pl.pallas_call(
    _post_body,
    out_shape=jax.ShapeDtypeStruct((1, G), jnp.float32),
)


def kernel(x, edge_index, batch, pos, Wg, bg, Wl, bl, Wp1, bp1, Wp2, bp2,
           Wa, ba, Wf, bf, W1, b1, W2, b2):
    src = edge_index[0]
    dst = edge_index[1]
    xT = x.T
    wp1_pack = jnp.broadcast_to(
        jnp.concatenate([Wp1.T, bp1[:, None]], axis=1)[:, :, None], (H, 4, L))

    degp = _deg_call(dst).reshape(NW, N)
    dinv, yT, xlT, ai, aj = _pre_call(
        degp, xT, Wg, Wl, bl[:, None], Wa[:H], Wa[H:], ba[:, None])
    tab = jnp.concatenate([
        yT.reshape(-1), xlT.reshape(-1), ai.reshape(-1), aj.reshape(-1),
        pos.T.reshape(-1), wp1_pack.reshape(-1)])
    ep = jnp.bitwise_or(jnp.left_shift(src, 14), dst)
    gT, axl, ah1, s0 = _main_call(ep, tab)
    out2 = _post_call(gT.reshape(H, N), axl.reshape(H, N), ah1.reshape(H, N),
                      s0[None, :], yT, dinv, batch[:, None],
                      Wp2, bp2[:, None], bg[:, None], Wf, bf[:, None],
                      W1, b1[:, None], W2, b2[:, None])
    return out2.reshape(G, 1)


# speedup vs baseline: 1.7089x; 1.7089x over previous
import jax
import jax.numpy as jnp
from jax import lax
from jax.experimental import pallas as pl
from jax.experimental.pallas import tpu as pltpu
from jax.experimental.pallas import tpu_sc as plsc

N = 10000
E = 320000
H = 128
G = 64
NC = 2
NS = 16
NW = NC * NS
CPW = H // NW
EPW = E // NW
CE = 4000
NCH = E // CE
ACE = 2000
ACH = EPW // ACE
L = 16

_MESH = dict(core_axis_name="c", subcore_axis_name="s", num_cores=NC,
             num_subcores=NS)


def _wid():
    return lax.axis_index("s") * NC + lax.axis_index("c")


def _unpack(e16):
    return lax.shift_right_logical(e16, 14), jnp.bitwise_and(e16, 16383)


def _deg_body(dst_hbm, degp_hbm, cnt, dbuf):
    w = _wid()
    zeros = jnp.zeros((L,), jnp.float32)
    ones = jnp.ones((L,), jnp.float32)

    @plsc.parallel_loop(0, N // L, unroll=8)
    def z(i):
        cnt[pl.ds(i * L, L)] = zeros

    pltpu.sync_copy(dst_hbm.at[pl.ds(w * EPW, EPW)], dbuf)

    @plsc.parallel_loop(0, EPW // L, unroll=8)
    def b(i):
        d16 = dbuf[pl.ds(i * L, L)]
        plsc.addupdate_scatter(cnt, [d16], ones)

    pltpu.sync_copy(cnt, degp_hbm.at[pl.ds(w * N, N)])


_deg_call = pl.kernel(
    _deg_body,
    out_type=jax.ShapeDtypeStruct((NW * N,), jnp.float32),
    mesh=plsc.VectorSubcoreMesh(**_MESH),
    compiler_params=pltpu.CompilerParams(needs_layout_passes=False),
    scratch_types=[
        pltpu.VMEM((N,), jnp.float32),
        pltpu.VMEM((EPW,), jnp.int32),
    ],
)


def _pre_body(degp, xT, Wg, Wl, bl2, wai2, waj2, ba2,
              dinv_o, yT_o, xlT_o, ai_o, aj_o):
    dot00 = lambda a, b: lax.dot_general(
        a, b, (((0,), (0,)), ((), ())), preferred_element_type=jnp.float32)
    deg = jnp.sum(degp[...], axis=0, keepdims=True) + 1.0
    dinv = lax.rsqrt(deg)
    dinv_o[...] = dinv
    xt = xT[...]
    yT_o[...] = dot00(Wg[...], xt) * dinv
    xlt = dot00(Wl[...], xt) + bl2[...]
    xlT_o[...] = xlt
    ai_o[...] = dot00(wai2[...], xlt) + ba2[...]
    aj_o[...] = dot00(waj2[...], xlt)


_pre_call = pl.pallas_call(
    _pre_body,
    out_shape=[
        jax.ShapeDtypeStruct((1, N), jnp.float32),
        jax.ShapeDtypeStruct((H, N), jnp.float32),
        jax.ShapeDtypeStruct((H, N), jnp.float32),
        jax.ShapeDtypeStruct((1, N), jnp.float32),
        jax.ShapeDtypeStruct((1, N), jnp.float32),
    ],
)


_OFF_Y = 0
_OFF_XL = H * N
_OFF_AI = 2 * H * N
_OFF_AJ = 2 * H * N + N
_OFF_POS = 2 * H * N + 2 * N
_OFF_WP1 = 2 * H * N + 5 * N
_TAB_SZ = _OFF_WP1 + H * 4 * L


def _attdp_body(ep_hbm, tab_hbm, att_o, dx_o, dy_o, dz_o, s0p_o,
                tabn, s0acc, ein0, ein1, oa0, oa1, ox0, ox1, oy0, oy1,
                oz0, oz1, semi0, semi1, semo0, semo1):
    w = _wid()
    base = w * EPW
    zeros = jnp.zeros((L,), jnp.float32)
    eins = (ein0, ein1)
    obufs = (((oa0, att_o), (ox0, dx_o), (oy0, dy_o), (oz0, dz_o)),
             ((oa1, att_o), (ox1, dx_o), (oy1, dy_o), (oz1, dz_o)))
    semis = (semi0, semi1)
    semos = (semo0, semo1)

    pltpu.sync_copy(tab_hbm.at[pl.ds(_OFF_AI, 5 * N)], tabn)

    @plsc.parallel_loop(0, N // L, unroll=8)
    def zs0(i):
        s0acc[pl.ds(i * L, L)] = zeros

    def issue_in(c, b):
        pltpu.async_copy(ep_hbm.at[pl.ds(base + c * ACE, ACE)], eins[b],
                         semis[b])

    def wait_in(b):
        pltpu.make_async_copy(ep_hbm.at[pl.ds(0, ACE)], eins[b],
                              semis[b]).wait()

    def issue_out(c, b):
        for buf, out in obufs[b]:
            pltpu.async_copy(buf, out.at[pl.ds(base + c * ACE, ACE)],
                             semos[b])

    def wait_out(b):
        for buf, out in obufs[b]:
            pltpu.make_async_copy(buf, out.at[pl.ds(0, ACE)],
                                  semos[b]).wait()

    def inner(b):
        oa, ox, oy, oz = (obufs[b][0][0], obufs[b][1][0], obufs[b][2][0],
                          obufs[b][3][0])

        @plsc.parallel_loop(0, ACE // L, unroll=8)
        def it(i):
            s16, d16 = _unpack(eins[b][pl.ds(i * L, L)])
            av = plsc.load_gather(tabn, [d16]) + plsc.load_gather(
                tabn, [s16 + N])
            att = 1.0 / (1.0 + jnp.exp(-av))
            dx = plsc.load_gather(tabn, [d16 + 2 * N]) - plsc.load_gather(
                tabn, [s16 + 2 * N])
            dy = plsc.load_gather(tabn, [d16 + 3 * N]) - plsc.load_gather(
                tabn, [s16 + 3 * N])
            dz = plsc.load_gather(tabn, [d16 + 4 * N]) - plsc.load_gather(
                tabn, [s16 + 4 * N])
            oa[pl.ds(i * L, L)] = att
            ox[pl.ds(i * L, L)] = dx
            oy[pl.ds(i * L, L)] = dy
            oz[pl.ds(i * L, L)] = dz
            plsc.addupdate_scatter(s0acc, [d16], att)

    issue_in(0, 0)
    issue_in(1, 1)
    for c in range(ACH):
        b = c % 2
        wait_in(b)
        if c >= 2:
            wait_out(b)
        inner(b)
        issue_out(c, b)
        if c + 2 < ACH:
            issue_in(c + 2, b)
    wait_out((ACH - 2) % 2)
    wait_out((ACH - 1) % 2)

    pltpu.sync_copy(s0acc, s0p_o.at[pl.ds(w * N, N)])


_attdp_call = pl.kernel(
    _attdp_body,
    out_type=[
        jax.ShapeDtypeStruct((E,), jnp.float32),
        jax.ShapeDtypeStruct((E,), jnp.float32),
        jax.ShapeDtypeStruct((E,), jnp.float32),
        jax.ShapeDtypeStruct((E,), jnp.float32),
        jax.ShapeDtypeStruct((NW * N,), jnp.float32),
    ],
    mesh=plsc.VectorSubcoreMesh(**_MESH),
    compiler_params=pltpu.CompilerParams(needs_layout_passes=False),
    scratch_types=[
        pltpu.VMEM((5 * N,), jnp.float32),
        pltpu.VMEM((N,), jnp.float32),
        pltpu.VMEM((ACE,), jnp.int32),
        pltpu.VMEM((ACE,), jnp.int32),
        pltpu.VMEM((ACE,), jnp.float32),
        pltpu.VMEM((ACE,), jnp.float32),
        pltpu.VMEM((ACE,), jnp.float32),
        pltpu.VMEM((ACE,), jnp.float32),
        pltpu.VMEM((ACE,), jnp.float32),
        pltpu.VMEM((ACE,), jnp.float32),
        pltpu.VMEM((ACE,), jnp.float32),
        pltpu.VMEM((ACE,), jnp.float32),
        pltpu.SemaphoreType.DMA,
        pltpu.SemaphoreType.DMA,
        pltpu.SemaphoreType.DMA,
        pltpu.SemaphoreType.DMA,
    ],
)


def _main_body(ep_hbm, tab_hbm, att_hbm, dx_hbm, dy_hbm, dz_hbm,
               gT_o, axl_o, ah1_o,
               tab4, acc, eb0, eb1, ab0, ab1, xb0, xb1, yb0, yb1,
               zb0, zb1, wv, sem0, sem1):
    w = _wid()
    w4 = w * CPW
    zeros = jnp.zeros((L,), jnp.float32)
    kns = [jnp.full((L,), k * N, jnp.int32) for k in range(4)]
    eb = (eb0, eb1)
    ab = (ab0, ab1)
    xb = (xb0, xb1)
    yb = (yb0, yb1)
    zb = (zb0, zb1)
    sems = (sem0, sem1)

    def zero_acc():
        @plsc.parallel_loop(0, CPW * N // L, unroll=8)
        def z(i):
            acc[pl.ds(i * L, L)] = zeros

    def edge_loop(streams, body16):
        def issue(c, b):
            for hbm, bufs in streams:
                pltpu.async_copy(hbm.at[pl.ds(c * CE, CE)], bufs[b], sems[b])

        def wait(b):
            for hbm, bufs in streams:
                pltpu.make_async_copy(hbm.at[pl.ds(0, CE)], bufs[b],
                                      sems[b]).wait()

        def inner(b):
            @plsc.parallel_loop(0, CE // L, unroll=8)
            def it(i):
                body16(*[bufs[b][pl.ds(i * L, L)] for _, bufs in streams])

        issue(0, 0)
        issue(1, 1)

        def outer(j, c):
            c0 = j * 2
            wait(0)
            inner(0)

            @pl.when(c0 + 2 < NCH)
            def _():
                issue(c0 + 2, 0)
            wait(1)
            inner(1)

            @pl.when(c0 + 3 < NCH)
            def _():
                issue(c0 + 3, 1)
            return c
        lax.fori_loop(0, NCH // 2, outer, 0)

    pltpu.sync_copy(tab_hbm.at[pl.ds(_OFF_Y + w4 * N, CPW * N)], tab4)
    zero_acc()

    def p_a(e16):
        s16, d16 = _unpack(e16)
        for k in range(4):
            v = plsc.load_gather(tab4, [s16 + kns[k]])
            plsc.addupdate_scatter(acc, [d16 + kns[k]], v)
    edge_loop([(ep_hbm, eb)], p_a)
    pltpu.sync_copy(acc, gT_o.at[pl.ds(w4 * N, CPW * N)])

    pltpu.sync_copy(tab_hbm.at[pl.ds(_OFF_XL + w4 * N, CPW * N)], tab4)
    zero_acc()

    def p_b1(e16, att):
        s16, d16 = _unpack(e16)
        for k in range(4):
            v = plsc.load_gather(tab4, [s16 + kns[k]]) * att
            plsc.addupdate_scatter(acc, [d16 + kns[k]], v)
    edge_loop([(ep_hbm, eb), (att_hbm, ab)], p_b1)
    pltpu.sync_copy(acc, axl_o.at[pl.ds(w4 * N, CPW * N)])

    pltpu.sync_copy(tab_hbm.at[pl.ds(_OFF_WP1 + w * 4 * L * 4, 4 * L * 4)], wv)
    zero_acc()
    wvv = [[wv[pl.ds((k * 4 + j) * L, L)] for j in range(4)] for k in range(4)]

    def p_b2(e16, att, dx, dy, dz):
        d16 = jnp.bitwise_and(e16, 16383)
        for k in range(4):
            h = jnp.maximum(
                dx * wvv[k][0] + dy * wvv[k][1] + dz * wvv[k][2] + wvv[k][3],
                0.0) * att
            plsc.addupdate_scatter(acc, [d16 + kns[k]], h)
    edge_loop([(ep_hbm, eb), (att_hbm, ab), (dx_hbm, xb), (dy_hbm, yb),
               (dz_hbm, zb)], p_b2)
    pltpu.sync_copy(acc, ah1_o.at[pl.ds(w4 * N, CPW * N)])


_main_call = pl.kernel(
    _main_body,
    out_type=[
        jax.ShapeDtypeStruct((H * N,), jnp.float32),
        jax.ShapeDtypeStruct((H * N,), jnp.float32),
        jax.ShapeDtypeStruct((H * N,), jnp.float32),
    ],
    mesh=plsc.VectorSubcoreMesh(**_MESH),
    compiler_params=pltpu.CompilerParams(needs_layout_passes=False),
    scratch_types=[
        pltpu.VMEM((CPW * N,), jnp.float32),
        pltpu.VMEM((CPW * N,), jnp.float32),
        pltpu.VMEM((CE,), jnp.int32),
        pltpu.VMEM((CE,), jnp.int32),
        pltpu.VMEM((CE,), jnp.float32),
        pltpu.VMEM((CE,), jnp.float32),
        pltpu.VMEM((CE,), jnp.float32),
        pltpu.VMEM((CE,), jnp.float32),
        pltpu.VMEM((CE,), jnp.float32),
        pltpu.VMEM((CE,), jnp.float32),
        pltpu.VMEM((CE,), jnp.float32),
        pltpu.VMEM((CE,), jnp.float32),
        pltpu.VMEM((4 * L * 4,), jnp.float32),
        pltpu.SemaphoreType.DMA,
        pltpu.SemaphoreType.DMA,
    ],
)


def _post_body(gT, axl, ah1, s0p, yT, dinv, batch2, Wp2, bp22, bg2,
               Wf, bf2, W1, b12, W2, b22, out_o):
    dot00 = lambda a, b: lax.dot_general(
        a, b, (((0,), (0,)), ((), ())), preferred_element_type=jnp.float32)
    s02 = jnp.sum(s0p[...], axis=0, keepdims=True)
    dv = dinv[...]
    globalT = jnp.maximum(dv * (gT[...] + yT[...]) + bg2[...], 0.0)
    localT = jnp.maximum(
        axl[...] + dot00(Wp2[...], ah1[...]) + bp22[...] * s02, 0.0)
    wf = Wf[...]
    fusedT = jnp.maximum(
        dot00(wf[:H], globalT) + dot00(wf[H:], localT) + bf2[...], 0.0)
    oh = (batch2[...] == lax.broadcasted_iota(jnp.int32, (N, G), 1)
          ).astype(jnp.float32)
    sumsT = jnp.dot(fusedT, oh, preferred_element_type=jnp.float32)
    cnts = jnp.sum(oh, axis=0, keepdims=True)
    pooledT = sumsT / jnp.maximum(cnts, 1.0)
    hT = jnp.maximum(dot00(W1[...], pooledT) + b12[...], 0.0)
    out_o[...] = dot00(W2[...], hT) + b22[...]


_post_call = pl.pallas_call(
    _post_body,
    out_shape=jax.ShapeDtypeStruct((1, G), jnp.float32),
)


def kernel(x, edge_index, batch, pos, Wg, bg, Wl, bl, Wp1, bp1, Wp2, bp2,
           Wa, ba, Wf, bf, W1, b1, W2, b2):
    src = edge_index[0]
    dst = edge_index[1]
    xT = x.T
    wp1_pack = jnp.broadcast_to(
        jnp.concatenate([Wp1.T, bp1[:, None]], axis=1)[:, :, None], (H, 4, L))

    degp = _deg_call(dst).reshape(NW, N)
    dinv, yT, xlT, ai, aj = _pre_call(
        degp, xT, Wg, Wl, bl[:, None], Wa[:H], Wa[H:], ba[:, None])
    tab = jnp.concatenate([
        yT.reshape(-1), xlT.reshape(-1), ai.reshape(-1), aj.reshape(-1),
        pos.T.reshape(-1), wp1_pack.reshape(-1)])
    ep = jnp.bitwise_or(jnp.left_shift(src, 14), dst)
    att, dxs, dys, dzs, s0p = _attdp_call(ep, tab)
    gT, axl, ah1 = _main_call(ep, tab, att, dxs, dys, dzs)
    out2 = _post_call(gT.reshape(H, N), axl.reshape(H, N), ah1.reshape(H, N),
                      s0p.reshape(NW, N), yT, dinv, batch[:, None],
                      Wp2, bp2[:, None], bg[:, None], Wf, bf[:, None],
                      W1, b1[:, None], W2, b2[:, None])
    return out2.reshape(G, 1)
